# Initial kernel scaffold; baseline (speedup 1.0000x reference)
import jax
import jax.numpy as jnp
from jax.experimental import pallas as pl


def kernel(x_user, x_item, edge_index_user_item, edge_index_item_user, params):
    raise NotImplementedError("write your pallas kernel here")


# speedup vs baseline: 16.0098x; 16.0098x over previous
import functools

import numpy as np
import jax
import jax.numpy as jnp
from jax import lax
from jax.experimental import pallas as pl
from jax.experimental.pallas import tpu as pltpu
from jax.experimental.pallas import tpu_sc as plsc

N_USER = 25000
N_ITEM = 25000
N_NODE = 25000
E_REL = 200000
C = 128
H = 4
D = C // H
NODE_TYPES = ["user", "item"]
RELATIONS = [("user", "u2i", "item"), ("item", "i2u", "user")]

NC = 2
NS = 16
NW = NC * NS
LANES = 16

EPAD = 204800
EPW = EPAD // NW
EPT = EPAD // NS
CH = 128
HALF0 = 12504
HALF1 = N_NODE - HALF0
DUMP = 12520
TBL = 12544

_SC_MESH = dict(core_axis_name="c", subcore_axis_name="s",
                num_cores=NC, num_subcores=NS)


def _mm_body(act, x_ref, w_ref, b_ref, o_ref):
    y = jnp.dot(x_ref[...], w_ref[...], preferred_element_type=jnp.float32)
    y = y + b_ref[...]
    if act == "relu":
        y = jnp.maximum(y, 0.0)
    o_ref[...] = y


def _mm(x, w, b, act=None, bm=1000):
    m, k = x.shape
    n = w.shape[1]
    return pl.pallas_call(
        functools.partial(_mm_body, act),
        grid=(m // bm,),
        in_specs=[pl.BlockSpec((bm, k), lambda i: (i, 0)),
                  pl.BlockSpec((k, n), lambda i: (0, 0)),
                  pl.BlockSpec((1, n), lambda i: (0, 0))],
        out_specs=pl.BlockSpec((bm, n), lambda i: (i, 0)),
        out_shape=jax.ShapeDtypeStruct((m, n), jnp.float32),
    )(x, w, b.reshape(1, n))


def _score_body(k_ref, q_ref, s_ref, sc_ref, mx_ref):
    t = k_ref[...] * q_ref[...]
    sc = jnp.dot(t, s_ref[...], preferred_element_type=jnp.float32)
    sc_ref[...] = sc
    cur = jnp.max(sc, axis=0, keepdims=True)

    @pl.when(pl.program_id(0) == 0)
    def _():
        mx_ref[...] = cur

    @pl.when(pl.program_id(0) != 0)
    def _():
        mx_ref[...] = jnp.maximum(mx_ref[...], cur)


def _scores(khat, qhat, smask, bm=2048):
    return pl.pallas_call(
        _score_body,
        grid=(EPAD // bm,),
        in_specs=[pl.BlockSpec((bm, C), lambda i: (i, 0)),
                  pl.BlockSpec((bm, C), lambda i: (i, 0)),
                  pl.BlockSpec((C, 8), lambda i: (0, 0))],
        out_specs=[pl.BlockSpec((bm, 8), lambda i: (i, 0)),
                   pl.BlockSpec((1, 8), lambda i: (0, 0))],
        out_shape=[jax.ShapeDtypeStruct((EPAD, 8), jnp.float32),
                   jax.ShapeDtypeStruct((1, 8), jnp.float32)],
    )(khat, qhat, smask)


def _msg_body(sc_ref, mx_ref, v_ref, r8_ref, p16_ref, msg_ref, e16_ref):
    e = jnp.exp(sc_ref[...] - mx_ref[...])
    e16_ref[...] = jnp.dot(e, p16_ref[...], preferred_element_type=jnp.float32)
    eexp = jnp.dot(e, r8_ref[...], preferred_element_type=jnp.float32)
    msg_ref[...] = v_ref[...] * eexp


def _messages(sc, mx, vhat, r8, p16, bm=2048):
    return pl.pallas_call(
        _msg_body,
        grid=(EPAD // bm,),
        in_specs=[pl.BlockSpec((bm, 8), lambda i: (i, 0)),
                  pl.BlockSpec((1, 8), lambda i: (0, 0)),
                  pl.BlockSpec((bm, C), lambda i: (i, 0)),
                  pl.BlockSpec((8, C), lambda i: (0, 0)),
                  pl.BlockSpec((8, 16), lambda i: (0, 0))],
        out_specs=[pl.BlockSpec((bm, C), lambda i: (i, 0)),
                   pl.BlockSpec((bm, 16), lambda i: (i, 0))],
        out_shape=[jax.ShapeDtypeStruct((EPAD, C), jnp.float32),
                   jax.ShapeDtypeStruct((EPAD, 16), jnp.float32)],
    )(sc, mx, vhat, r8, p16)


def _out_body(agg_ref, s_ref, p2_ref, wa_ref, ba_ref, xc_ref, xp_ref, o_ref):
    denom = jnp.maximum(jnp.dot(s_ref[...], p2_ref[...],
                                preferred_element_type=jnp.float32), 1e-30)
    u = agg_ref[...] / denom
    g = jax.nn.gelu(u)
    y = jnp.dot(g, wa_ref[...], preferred_element_type=jnp.float32)
    o_ref[...] = y + ba_ref[...] + xc_ref[...] * xp_ref[...]


def _out_proj(agg, s16, p2, wa, ba, xc, xprev, bm=1000):
    return pl.pallas_call(
        _out_body,
        grid=(N_NODE // bm,),
        in_specs=[pl.BlockSpec((bm, C), lambda i: (i, 0)),
                  pl.BlockSpec((bm, 16), lambda i: (i, 0)),
                  pl.BlockSpec((16, C), lambda i: (0, 0)),
                  pl.BlockSpec((C, C), lambda i: (0, 0)),
                  pl.BlockSpec((1, C), lambda i: (0, 0)),
                  pl.BlockSpec((1, C), lambda i: (0, 0)),
                  pl.BlockSpec((bm, C), lambda i: (i, 0))],
        out_specs=pl.BlockSpec((bm, C), lambda i: (i, 0)),
        out_shape=jax.ShapeDtypeStruct((N_NODE, C), jnp.float32),
    )(agg, s16, p2, wa, ba, xc, xprev)


def _heads_body(xu_ref, xi_ref, w0_ref, b0_ref, w1_ref, b1_ref,
                wh0_ref, bh0_ref, wh1_ref, bh1_ref, wh2_ref, bh2_ref,
                wv0_ref, bv0_ref, wv1_ref, bv1_ref,
                o0_ref, o1_ref, o2_ref, ov_ref, acc_ref):
    i = pl.program_id(0)
    su = jnp.sum(xu_ref[...], axis=0, keepdims=True)
    si = jnp.sum(xi_ref[...], axis=0, keepdims=True)
    cur = jnp.concatenate([su, si], axis=1)

    @pl.when(i == 0)
    def _():
        acc_ref[0:1, :] = cur

    @pl.when(i > 0)
    def _():
        acc_ref[0:1, :] = acc_ref[0:1, :] + cur

    @pl.when(i == pl.num_programs(0) - 1)
    def _():
        pooled = acc_ref[0:1, :] * (1.0 / N_NODE)

        def lin(v, w_ref, b_ref, act=False):
            y = jnp.dot(v, w_ref[...], preferred_element_type=jnp.float32)
            y = y + b_ref[...]
            return jnp.maximum(y, 0.0) if act else y

        h = lin(pooled, w0_ref, b0_ref, act=True)
        h = lin(h, w1_ref, b1_ref, act=True)
        o0_ref[...] = lin(h, wh0_ref, bh0_ref)
        o1_ref[...] = lin(h, wh1_ref, bh1_ref)
        o2_ref[...] = lin(h, wh2_ref, bh2_ref)
        vh = lin(h, wv0_ref, bv0_ref, act=True)
        ov_ref[...] = lin(vh, wv1_ref, bv1_ref)


def _pool_heads(xu, xi, params, bm=1000):
    sh = params["shared"]
    hd = params["heads"]
    vl = params["value"]
    full = lambda s: pl.BlockSpec(s, lambda i: tuple(0 for _ in s))
    args = [xu, xi,
            sh[0]["W"], sh[0]["b"].reshape(1, -1),
            sh[1]["W"], sh[1]["b"].reshape(1, -1),
            hd[0]["W"], hd[0]["b"].reshape(1, -1),
            hd[1]["W"], hd[1]["b"].reshape(1, -1),
            hd[2]["W"], hd[2]["b"].reshape(1, -1),
            vl[0]["W"], vl[0]["b"].reshape(1, -1),
            vl[1]["W"], vl[1]["b"].reshape(1, -1)]
    in_specs = [pl.BlockSpec((bm, C), lambda i: (i, 0)),
                pl.BlockSpec((bm, C), lambda i: (i, 0))]
    for a in args[2:]:
        in_specs.append(full(a.shape))
    out_shapes = [jax.ShapeDtypeStruct((1, 8), jnp.float32),
                  jax.ShapeDtypeStruct((1, 8), jnp.float32),
                  jax.ShapeDtypeStruct((1, 4), jnp.float32),
                  jax.ShapeDtypeStruct((1, 1), jnp.float32)]
    out_specs = [full(s.shape) for s in out_shapes]
    return pl.pallas_call(
        _heads_body,
        grid=(N_NODE // bm,),
        in_specs=in_specs,
        out_specs=out_specs,
        out_shape=out_shapes,
        scratch_shapes=[pltpu.VMEM((8, 2 * C), jnp.float32)],
    )(*args)


def _gather_pair_body(ta, ia, tb, ib, oa, ob, iva, rva, ivb, rvb, sa, sb):
    wid = lax.axis_index("s") * NC + lax.axis_index("c")

    def body(i, carry):
        base = wid * EPW + i * CH
        pltpu.sync_copy(ia.at[pl.ds(base, CH)], iva)
        pltpu.sync_copy(ib.at[pl.ds(base, CH)], ivb)
        da = pltpu.async_copy(ta.at[iva], rva, sa)
        db = pltpu.async_copy(tb.at[ivb], rvb, sb)
        da.wait()
        db.wait()
        pltpu.sync_copy(rva, oa.at[pl.ds(base, CH)])
        pltpu.sync_copy(rvb, ob.at[pl.ds(base, CH)])
        return carry

    lax.fori_loop(0, EPW // CH, body, 0)


def _gather_pair(ta, ia, tb, ib):
    k = pl.kernel(
        _gather_pair_body,
        out_type=(jax.ShapeDtypeStruct((EPAD, C), jnp.float32),
                  jax.ShapeDtypeStruct((EPAD, C), jnp.float32)),
        mesh=plsc.VectorSubcoreMesh(**_SC_MESH),
        scratch_types=[pltpu.VMEM((CH,), jnp.int32),
                       pltpu.VMEM((CH, C), jnp.float32),
                       pltpu.VMEM((CH,), jnp.int32),
                       pltpu.VMEM((CH, C), jnp.float32),
                       pltpu.SemaphoreType.DMA,
                       pltpu.SemaphoreType.DMA],
    )
    return k(ta, ia, tb, ib)


def _gather_one_body(ta, ia, oa, iva, rva, sa):
    wid = lax.axis_index("s") * NC + lax.axis_index("c")

    def body(i, carry):
        base = wid * EPW + i * CH
        pltpu.sync_copy(ia.at[pl.ds(base, CH)], iva)
        pltpu.async_copy(ta.at[iva], rva, sa).wait()
        pltpu.sync_copy(rva, oa.at[pl.ds(base, CH)])
        return carry

    lax.fori_loop(0, EPW // CH, body, 0)


def _gather_one(ta, ia):
    k = pl.kernel(
        _gather_one_body,
        out_type=jax.ShapeDtypeStruct((EPAD, C), jnp.float32),
        mesh=plsc.VectorSubcoreMesh(**_SC_MESH),
        scratch_types=[pltpu.VMEM((CH,), jnp.int32),
                       pltpu.VMEM((CH, C), jnp.float32),
                       pltpu.SemaphoreType.DMA],
    )
    return k(ta, ia)


def _make_scatter_body(ncol):
    def _scatter_body(msg, dst, agg, dbuf, lbuf, mbuf, tmsg):
        cid = lax.axis_index("c")
        sid = lax.axis_index("s")
        zv = jnp.zeros((LANES,), jnp.float32)

        for rr in range(CH):
            for cc in range(ncol // LANES):
                mbuf[rr, pl.ds(cc * LANES, LANES)] = zv
        zr = TBL // NS
        for t in range(zr // CH):
            pltpu.sync_copy(mbuf, tmsg.at[pl.ds(sid * zr + t * CH, CH)])
        rem = zr % CH
        pltpu.sync_copy(mbuf.at[pl.ds(0, rem)],
                        tmsg.at[pl.ds(sid * zr + (zr // CH) * CH, rem)])
        plsc.subcore_barrier()

        lo = cid * HALF0
        halfc = HALF0 - (HALF0 - HALF1) * cid

        def body(i, carry):
            base = sid * EPT + i * CH
            pltpu.sync_copy(dst.at[pl.ds(base, CH)], dbuf)
            pltpu.sync_copy(msg.at[pl.ds(base, CH)], mbuf)
            for j in range(CH // LANES):
                v = dbuf[pl.ds(j * LANES, LANES)]
                l = v - lo
                ok = (l >= 0) & (l < halfc)
                lbuf[pl.ds(j * LANES, LANES)] = jnp.where(ok, l, DUMP)
            pltpu.sync_copy(mbuf, tmsg.at[lbuf], add=True)
            return carry

        lax.fori_loop(0, EPT // CH, body, 0)
        plsc.subcore_barrier()

        nfull = 97

        def cpy(t, carry):
            ci = sid + NS * t

            @pl.when(ci < nfull)
            def _():
                off = ci * CH
                pltpu.sync_copy(tmsg.at[pl.ds(off, CH)], mbuf)
                pltpu.sync_copy(mbuf, agg.at[pl.ds(lo + off, CH)])
            return carry

        lax.fori_loop(0, (nfull + NS - 1) // NS, cpy, 0)

        roff = nfull * CH

        @pl.when((sid == NS - 1) & (cid == 0))
        def _():
            pltpu.sync_copy(tmsg.at[pl.ds(roff, 88)], mbuf.at[pl.ds(0, 88)])
            pltpu.sync_copy(mbuf.at[pl.ds(0, 88)], agg.at[pl.ds(roff, 88)])

        @pl.when((sid == NS - 1) & (cid == 1))
        def _():
            pltpu.sync_copy(tmsg.at[pl.ds(roff, 80)], mbuf.at[pl.ds(0, 80)])
            pltpu.sync_copy(mbuf.at[pl.ds(0, 80)],
                            agg.at[pl.ds(HALF0 + roff, 80)])

    return _scatter_body


def _scatter(rows, dst, ncol):
    k = pl.kernel(
        _make_scatter_body(ncol),
        out_type=jax.ShapeDtypeStruct((N_NODE, ncol), jnp.float32),
        mesh=plsc.VectorSubcoreMesh(**_SC_MESH),
        scratch_types=[pltpu.VMEM((CH,), jnp.int32),
                       pltpu.VMEM((CH,), jnp.int32),
                       pltpu.VMEM((CH, ncol), jnp.float32),
                       pltpu.VMEM_SHARED((TBL, ncol), jnp.float32)],
    )
    return k(rows, dst)


def _block_diag(a):
    out = jnp.zeros((C, C), dtype=jnp.float32)
    for h in range(H):
        out = lax.dynamic_update_slice(out, a[h], (h * D, h * D))
    return out


def kernel(x_user, x_item, edge_index_user_item, edge_index_item_user, params):
    edge = {"u2i": edge_index_user_item, "i2u": edge_index_item_user}

    smask = np.zeros((C, 8), np.float32)
    for h in range(H):
        smask[h * D:(h + 1) * D, h] = 1.0
    smask = jnp.asarray(smask)
    r8 = np.zeros((8, C), np.float32)
    for h in range(H):
        r8[h, h * D:(h + 1) * D] = 1.0
    r8 = jnp.asarray(r8)
    p16 = np.zeros((8, 16), np.float32)
    for h in range(H):
        p16[h, h] = 1.0
    p16 = jnp.asarray(p16)
    p2 = np.zeros((16, C), np.float32)
    for h in range(H):
        p2[h, h * D:(h + 1) * D] = 1.0
    p2 = jnp.asarray(p2)

    pad_i = jnp.zeros((EPAD - E_REL,), jnp.int32)
    pad_d = jnp.full((EPAD - E_REL,), N_NODE, jnp.int32)
    src_pad = {r: jnp.concatenate([edge[r][0], pad_i]) for (_, r, _) in RELATIONS}
    dst_pad = {r: jnp.concatenate([edge[r][1], pad_d]) for (_, r, _) in RELATIONS}

    x = {"user": _mm(x_user, params["lin_in"]["user"]["W"],
                     params["lin_in"]["user"]["b"], act="relu"),
         "item": _mm(x_item, params["lin_in"]["item"]["W"],
                     params["lin_in"]["item"]["b"], act="relu")}

    for layer in params["layers"]:
        newx = {}
        for (s, r, d) in RELATIONS:
            rp = layer["rel"][r]
            colscale = jnp.repeat(rp["p_rel"], D) * (1.0 / np.sqrt(D))
            ak = _block_diag(rp["a_rel"]) * colscale[None, :]
            mv = _block_diag(rp["m_rel"])
            wk = layer["k"][s]["W"] @ ak
            bk = layer["k"][s]["b"] @ ak
            wv = layer["v"][s]["W"] @ mv
            bv = layer["v"][s]["b"] @ mv

            krel = _mm(x[s], wk, bk)
            vrel = _mm(x[s], wv, bv)
            q = _mm(x[d], layer["q"][d]["W"], layer["q"][d]["b"])

            khat, qhat = _gather_pair(krel, src_pad[r], q, dst_pad[r])
            vhat = _gather_one(vrel, src_pad[r])

            sc, mx = _scores(khat, qhat, smask)
            msgs, e16 = _messages(sc, mx, vhat, r8, p16)
            agg = _scatter(msgs, dst_pad[r], C)
            s16 = _scatter(e16, dst_pad[r], 16)

            beta = jax.nn.sigmoid(layer["skip"][d])
            wa = layer["a"][d]["W"] * beta
            ba = (layer["a"][d]["b"] * beta).reshape(1, C)
            xc = jnp.full((1, C), 1.0 - beta, jnp.float32)
            newx[d] = _out_proj(agg, s16, p2, wa, ba, xc, x[d])
        x = newx

    o0, o1, o2, ov = _pool_heads(x["user"], x["item"], params)
    return (o0, o1, o2, ov)


# speedup vs baseline: 17.1966x; 1.0741x over previous
import functools

import numpy as np
import jax
import jax.numpy as jnp
from jax import lax
from jax.experimental import pallas as pl
from jax.experimental.pallas import tpu as pltpu
from jax.experimental.pallas import tpu_sc as plsc

N_USER = 25000
N_ITEM = 25000
N_NODE = 25000
E_REL = 200000
C = 128
H = 4
D = C // H
NODE_TYPES = ["user", "item"]
RELATIONS = [("user", "u2i", "item"), ("item", "i2u", "user")]

NC = 2
NS = 16
NW = NC * NS
LANES = 16

EPAD = 204800
EPW = EPAD // NW
EPT = EPAD // NS
CH = 128
HALF0 = 12504
HALF1 = N_NODE - HALF0
DUMP = 12520
TBL = 12544

_SC_MESH = dict(core_axis_name="c", subcore_axis_name="s",
                num_cores=NC, num_subcores=NS)


def _mm_body(act, x_ref, w_ref, b_ref, o_ref):
    y = jnp.dot(x_ref[...], w_ref[...], preferred_element_type=jnp.float32)
    y = y + b_ref[...]
    if act == "relu":
        y = jnp.maximum(y, 0.0)
    o_ref[...] = y


def _mm(x, w, b, act=None, bm=1000):
    m, k = x.shape
    n = w.shape[1]
    return pl.pallas_call(
        functools.partial(_mm_body, act),
        grid=(m // bm,),
        in_specs=[pl.BlockSpec((bm, k), lambda i: (i, 0)),
                  pl.BlockSpec((k, n), lambda i: (0, 0)),
                  pl.BlockSpec((1, n), lambda i: (0, 0))],
        out_specs=pl.BlockSpec((bm, n), lambda i: (i, 0)),
        out_shape=jax.ShapeDtypeStruct((m, n), jnp.float32),
    )(x, w, b.reshape(1, n))


def _score_body(k_ref, q_ref, s_ref, sc_ref, mx_ref):
    t = k_ref[...] * q_ref[...]
    sc = jnp.dot(t, s_ref[...], preferred_element_type=jnp.float32)
    sc_ref[...] = sc
    cur = jnp.max(sc, axis=0, keepdims=True)

    @pl.when(pl.program_id(0) == 0)
    def _():
        mx_ref[...] = cur

    @pl.when(pl.program_id(0) != 0)
    def _():
        mx_ref[...] = jnp.maximum(mx_ref[...], cur)


def _scores(khat, qhat, smask, bm=2048):
    return pl.pallas_call(
        _score_body,
        grid=(EPAD // bm,),
        in_specs=[pl.BlockSpec((bm, C), lambda i: (i, 0)),
                  pl.BlockSpec((bm, C), lambda i: (i, 0)),
                  pl.BlockSpec((C, 8), lambda i: (0, 0))],
        out_specs=[pl.BlockSpec((bm, 8), lambda i: (i, 0)),
                   pl.BlockSpec((1, 8), lambda i: (0, 0))],
        out_shape=[jax.ShapeDtypeStruct((EPAD, 8), jnp.float32),
                   jax.ShapeDtypeStruct((1, 8), jnp.float32)],
    )(khat, qhat, smask)


def _msg_body(sc_ref, mx_ref, v_ref, r8_ref, p16_ref, msg_ref, e16_ref):
    e = jnp.exp(sc_ref[...] - mx_ref[...])
    e16_ref[...] = jnp.dot(e, p16_ref[...], preferred_element_type=jnp.float32)
    eexp = jnp.dot(e, r8_ref[...], preferred_element_type=jnp.float32)
    msg_ref[...] = v_ref[...] * eexp


def _messages(sc, mx, vhat, r8, p16, bm=2048):
    return pl.pallas_call(
        _msg_body,
        grid=(EPAD // bm,),
        in_specs=[pl.BlockSpec((bm, 8), lambda i: (i, 0)),
                  pl.BlockSpec((1, 8), lambda i: (0, 0)),
                  pl.BlockSpec((bm, C), lambda i: (i, 0)),
                  pl.BlockSpec((8, C), lambda i: (0, 0)),
                  pl.BlockSpec((8, 16), lambda i: (0, 0))],
        out_specs=[pl.BlockSpec((bm, C), lambda i: (i, 0)),
                   pl.BlockSpec((bm, 16), lambda i: (i, 0))],
        out_shape=[jax.ShapeDtypeStruct((EPAD, C), jnp.float32),
                   jax.ShapeDtypeStruct((EPAD, 16), jnp.float32)],
    )(sc, mx, vhat, r8, p16)


def _out_body(agg_ref, s_ref, p2_ref, wa_ref, ba_ref, xc_ref, xp_ref, o_ref):
    denom = jnp.maximum(jnp.dot(s_ref[...], p2_ref[...],
                                preferred_element_type=jnp.float32), 1e-30)
    u = agg_ref[...] / denom
    g = jax.nn.gelu(u)
    y = jnp.dot(g, wa_ref[...], preferred_element_type=jnp.float32)
    o_ref[...] = y + ba_ref[...] + xc_ref[...] * xp_ref[...]


def _out_proj(agg, s16, p2, wa, ba, xc, xprev, bm=1000):
    return pl.pallas_call(
        _out_body,
        grid=(N_NODE // bm,),
        in_specs=[pl.BlockSpec((bm, C), lambda i: (i, 0)),
                  pl.BlockSpec((bm, 16), lambda i: (i, 0)),
                  pl.BlockSpec((16, C), lambda i: (0, 0)),
                  pl.BlockSpec((C, C), lambda i: (0, 0)),
                  pl.BlockSpec((1, C), lambda i: (0, 0)),
                  pl.BlockSpec((1, C), lambda i: (0, 0)),
                  pl.BlockSpec((bm, C), lambda i: (i, 0))],
        out_specs=pl.BlockSpec((bm, C), lambda i: (i, 0)),
        out_shape=jax.ShapeDtypeStruct((N_NODE, C), jnp.float32),
    )(agg, s16, p2, wa, ba, xc, xprev)


def _heads_body(xu_ref, xi_ref, w0_ref, b0_ref, w1_ref, b1_ref,
                wh0_ref, bh0_ref, wh1_ref, bh1_ref, wh2_ref, bh2_ref,
                wv0_ref, bv0_ref, wv1_ref, bv1_ref,
                o0_ref, o1_ref, o2_ref, ov_ref, acc_ref):
    i = pl.program_id(0)
    su = jnp.sum(xu_ref[...], axis=0, keepdims=True)
    si = jnp.sum(xi_ref[...], axis=0, keepdims=True)
    cur = jnp.concatenate([su, si], axis=1)

    @pl.when(i == 0)
    def _():
        acc_ref[0:1, :] = cur

    @pl.when(i > 0)
    def _():
        acc_ref[0:1, :] = acc_ref[0:1, :] + cur

    @pl.when(i == pl.num_programs(0) - 1)
    def _():
        pooled = acc_ref[0:1, :] * (1.0 / N_NODE)

        def lin(v, w_ref, b_ref, act=False):
            y = jnp.dot(v, w_ref[...], preferred_element_type=jnp.float32)
            y = y + b_ref[...]
            return jnp.maximum(y, 0.0) if act else y

        h = lin(pooled, w0_ref, b0_ref, act=True)
        h = lin(h, w1_ref, b1_ref, act=True)
        o0_ref[...] = lin(h, wh0_ref, bh0_ref)
        o1_ref[...] = lin(h, wh1_ref, bh1_ref)
        o2_ref[...] = lin(h, wh2_ref, bh2_ref)
        vh = lin(h, wv0_ref, bv0_ref, act=True)
        ov_ref[...] = lin(vh, wv1_ref, bv1_ref)


def _pool_heads(xu, xi, params, bm=1000):
    sh = params["shared"]
    hd = params["heads"]
    vl = params["value"]
    full = lambda s: pl.BlockSpec(s, lambda i: tuple(0 for _ in s))
    args = [xu, xi,
            sh[0]["W"], sh[0]["b"].reshape(1, -1),
            sh[1]["W"], sh[1]["b"].reshape(1, -1),
            hd[0]["W"], hd[0]["b"].reshape(1, -1),
            hd[1]["W"], hd[1]["b"].reshape(1, -1),
            hd[2]["W"], hd[2]["b"].reshape(1, -1),
            vl[0]["W"], vl[0]["b"].reshape(1, -1),
            vl[1]["W"], vl[1]["b"].reshape(1, -1)]
    in_specs = [pl.BlockSpec((bm, C), lambda i: (i, 0)),
                pl.BlockSpec((bm, C), lambda i: (i, 0))]
    for a in args[2:]:
        in_specs.append(full(a.shape))
    out_shapes = [jax.ShapeDtypeStruct((1, 8), jnp.float32),
                  jax.ShapeDtypeStruct((1, 8), jnp.float32),
                  jax.ShapeDtypeStruct((1, 4), jnp.float32),
                  jax.ShapeDtypeStruct((1, 1), jnp.float32)]
    out_specs = [full(s.shape) for s in out_shapes]
    return pl.pallas_call(
        _heads_body,
        grid=(N_NODE // bm,),
        in_specs=in_specs,
        out_specs=out_specs,
        out_shape=out_shapes,
        scratch_shapes=[pltpu.VMEM((8, 2 * C), jnp.float32)],
    )(*args)


NCHG = EPW // CH
NCHS = EPT // CH
NBUF = 3


def _gather3_body(tk, ik, tq, iq, tv, okh, oqh, ovh,
                  i1, i2, r1, r2, s1, s2):
    wid = lax.axis_index("s") * NC + lax.axis_index("c")

    def phase(tA, iA, cA, oA, tB, iB, cB, oB):
        baseA = wid * EPW + cA * CH
        baseB = wid * EPW + cB * CH
        pltpu.sync_copy(iA.at[pl.ds(baseA, CH)], i1)
        pltpu.sync_copy(iB.at[pl.ds(baseB, CH)], i2)
        dA = pltpu.async_copy(tA.at[i1], r1, s1)
        dB = pltpu.async_copy(tB.at[i2], r2, s2)
        dA.wait()
        dB.wait()
        pltpu.sync_copy(r1, oA.at[pl.ds(baseA, CH)])
        pltpu.sync_copy(r2, oB.at[pl.ds(baseB, CH)])

    def body(g, carry):
        c0 = 2 * g
        c1 = 2 * g + 1
        phase(tk, ik, c0, okh, tq, iq, c0, oqh)
        phase(tv, ik, c0, ovh, tk, ik, c1, okh)
        phase(tq, iq, c1, oqh, tv, ik, c1, ovh)
        return carry

    lax.fori_loop(0, NCHG // 2, body, 0)


def _gather3(tk, ik, tq, iq, tv):
    k = pl.kernel(
        _gather3_body,
        out_type=(jax.ShapeDtypeStruct((EPAD, C), jnp.float32),
                  jax.ShapeDtypeStruct((EPAD, C), jnp.float32),
                  jax.ShapeDtypeStruct((EPAD, C), jnp.float32)),
        mesh=plsc.VectorSubcoreMesh(**_SC_MESH),
        scratch_types=[pltpu.VMEM((CH,), jnp.int32),
                       pltpu.VMEM((CH,), jnp.int32),
                       pltpu.VMEM((CH, C), jnp.float32),
                       pltpu.VMEM((CH, C), jnp.float32),
                       pltpu.SemaphoreType.DMA,
                       pltpu.SemaphoreType.DMA],
    )
    return k(tk, ik, tq, iq, tv)


SCH = 128


def _make_scatter_body(ncol):
    def _scatter_body(msg, dst, agg, dbuf, lbuf, mbuf, tmsg):
        cid = lax.axis_index("c")
        sid = lax.axis_index("s")
        zv = jnp.zeros((LANES,), jnp.float32)

        for rr in range(SCH):
            for cc in range(ncol // LANES):
                mbuf[rr, pl.ds(cc * LANES, LANES)] = zv
        zr = TBL // NS
        for t in range(zr // SCH):
            pltpu.sync_copy(mbuf, tmsg.at[pl.ds(sid * zr + t * SCH, SCH)])
        rem = zr % SCH
        if rem:
            pltpu.sync_copy(mbuf.at[pl.ds(0, rem)],
                            tmsg.at[pl.ds(sid * zr + (zr // SCH) * SCH, rem)])
        plsc.subcore_barrier()

        lo = cid * HALF0
        halfc = HALF0 - (HALF0 - HALF1) * cid

        def body(i, carry):
            base = sid * EPT + i * SCH
            pltpu.sync_copy(dst.at[pl.ds(base, SCH)], dbuf)
            pltpu.sync_copy(msg.at[pl.ds(base, SCH)], mbuf)
            for j in range(SCH // LANES):
                v = dbuf[pl.ds(j * LANES, LANES)]
                l = v - lo
                ok = (l >= 0) & (l < halfc)
                lbuf[pl.ds(j * LANES, LANES)] = jnp.where(ok, l, DUMP)
            pltpu.sync_copy(mbuf, tmsg.at[lbuf], add=True)
            return carry

        lax.fori_loop(0, EPT // SCH, body, 0)
        plsc.subcore_barrier()

        nfull0 = HALF0 // SCH
        nfull1 = HALF1 // SCH
        nfullc = nfull0 - (nfull0 - nfull1) * cid

        def cpy(t, carry):
            ci = sid + NS * t

            @pl.when(ci < nfullc)
            def _():
                off = ci * SCH
                pltpu.sync_copy(tmsg.at[pl.ds(off, SCH)], mbuf)
                pltpu.sync_copy(mbuf, agg.at[pl.ds(lo + off, SCH)])
            return carry

        lax.fori_loop(0, (max(nfull0, nfull1) + NS - 1) // NS, cpy, 0)

        rem0 = HALF0 % SCH
        rem1 = HALF1 % SCH

        if rem0:
            @pl.when((sid == NS - 1) & (cid == 0))
            def _():
                roff = nfull0 * SCH
                pltpu.sync_copy(tmsg.at[pl.ds(roff, rem0)],
                                mbuf.at[pl.ds(0, rem0)])
                pltpu.sync_copy(mbuf.at[pl.ds(0, rem0)],
                                agg.at[pl.ds(roff, rem0)])

        if rem1:
            @pl.when((sid == NS - 1) & (cid == 1))
            def _():
                roff = nfull1 * SCH
                pltpu.sync_copy(tmsg.at[pl.ds(roff, rem1)],
                                mbuf.at[pl.ds(0, rem1)])
                pltpu.sync_copy(mbuf.at[pl.ds(0, rem1)],
                                agg.at[pl.ds(HALF0 + roff, rem1)])

    return _scatter_body


def _scatter(rows, dst, ncol):
    k = pl.kernel(
        _make_scatter_body(ncol),
        out_type=jax.ShapeDtypeStruct((N_NODE, ncol), jnp.float32),
        mesh=plsc.VectorSubcoreMesh(**_SC_MESH),
        scratch_types=[pltpu.VMEM((SCH,), jnp.int32),
                       pltpu.VMEM((SCH,), jnp.int32),
                       pltpu.VMEM((SCH, ncol), jnp.float32),
                       pltpu.VMEM_SHARED((TBL, ncol), jnp.float32)],
    )
    return k(rows, dst)


def _block_diag(a):
    out = jnp.zeros((C, C), dtype=jnp.float32)
    for h in range(H):
        out = lax.dynamic_update_slice(out, a[h], (h * D, h * D))
    return out


def kernel(x_user, x_item, edge_index_user_item, edge_index_item_user, params):
    edge = {"u2i": edge_index_user_item, "i2u": edge_index_item_user}

    smask = np.zeros((C, 8), np.float32)
    for h in range(H):
        smask[h * D:(h + 1) * D, h] = 1.0
    smask = jnp.asarray(smask)
    r8 = np.zeros((8, C), np.float32)
    for h in range(H):
        r8[h, h * D:(h + 1) * D] = 1.0
    r8 = jnp.asarray(r8)
    p16 = np.zeros((8, 16), np.float32)
    for h in range(H):
        p16[h, h] = 1.0
    p16 = jnp.asarray(p16)
    p2 = np.zeros((16, C), np.float32)
    for h in range(H):
        p2[h, h * D:(h + 1) * D] = 1.0
    p2 = jnp.asarray(p2)

    pad_i = jnp.zeros((EPAD - E_REL,), jnp.int32)
    pad_d = jnp.full((EPAD - E_REL,), N_NODE, jnp.int32)
    src_pad = {r: jnp.concatenate([edge[r][0], pad_i]) for (_, r, _) in RELATIONS}
    dst_pad = {r: jnp.concatenate([edge[r][1], pad_d]) for (_, r, _) in RELATIONS}

    x = {"user": _mm(x_user, params["lin_in"]["user"]["W"],
                     params["lin_in"]["user"]["b"], act="relu"),
         "item": _mm(x_item, params["lin_in"]["item"]["W"],
                     params["lin_in"]["item"]["b"], act="relu")}

    for layer in params["layers"]:
        newx = {}
        for (s, r, d) in RELATIONS:
            rp = layer["rel"][r]
            colscale = jnp.repeat(rp["p_rel"], D) * (1.0 / np.sqrt(D))
            ak = _block_diag(rp["a_rel"]) * colscale[None, :]
            mv = _block_diag(rp["m_rel"])
            wk = layer["k"][s]["W"] @ ak
            bk = layer["k"][s]["b"] @ ak
            wv = layer["v"][s]["W"] @ mv
            bv = layer["v"][s]["b"] @ mv

            krel = _mm(x[s], wk, bk)
            vrel = _mm(x[s], wv, bv)
            q = _mm(x[d], layer["q"][d]["W"], layer["q"][d]["b"])

            khat, qhat, vhat = _gather3(krel, src_pad[r], q, dst_pad[r],
                                        vrel)

            sc, mx = _scores(khat, qhat, smask)
            msgs, e16 = _messages(sc, mx, vhat, r8, p16)
            agg = _scatter(msgs, dst_pad[r], C)
            s16 = _scatter(e16, dst_pad[r], 16)

            beta = jax.nn.sigmoid(layer["skip"][d])
            wa = layer["a"][d]["W"] * beta
            ba = (layer["a"][d]["b"] * beta).reshape(1, C)
            xc = jnp.full((1, C), 1.0 - beta, jnp.float32)
            newx[d] = _out_proj(agg, s16, p2, wa, ba, xc, x[d])
        x = newx

    o0, o1, o2, ov = _pool_heads(x["user"], x["item"], params)
    return (o0, o1, o2, ov)


# speedup vs baseline: 18.3083x; 1.0647x over previous
import functools

import numpy as np
import jax
import jax.numpy as jnp
from jax import lax
from jax.experimental import pallas as pl
from jax.experimental.pallas import tpu as pltpu
from jax.experimental.pallas import tpu_sc as plsc

N_USER = 25000
N_ITEM = 25000
N_NODE = 25000
E_REL = 200000
C = 128
H = 4
D = C // H
NODE_TYPES = ["user", "item"]
RELATIONS = [("user", "u2i", "item"), ("item", "i2u", "user")]

NC = 2
NS = 16
NW = NC * NS
LANES = 16

EPAD = 204800
EPW = EPAD // NW
EPT = EPAD // NS
CH = 128
HALF0 = 12504
HALF1 = N_NODE - HALF0
DUMP = 12520
TBL = 12544

_SC_MESH = dict(core_axis_name="c", subcore_axis_name="s",
                num_cores=NC, num_subcores=NS)


def _mm_body(act, x_ref, w_ref, b_ref, o_ref):
    y = jnp.dot(x_ref[...], w_ref[...], preferred_element_type=jnp.float32)
    y = y + b_ref[...]
    if act == "relu":
        y = jnp.maximum(y, 0.0)
    o_ref[...] = y


def _mm(x, w, b, act=None, bm=1000):
    m, k = x.shape
    n = w.shape[1]
    return pl.pallas_call(
        functools.partial(_mm_body, act),
        grid=(m // bm,),
        in_specs=[pl.BlockSpec((bm, k), lambda i: (i, 0)),
                  pl.BlockSpec((k, n), lambda i: (0, 0)),
                  pl.BlockSpec((1, n), lambda i: (0, 0))],
        out_specs=pl.BlockSpec((bm, n), lambda i: (i, 0)),
        out_shape=jax.ShapeDtypeStruct((m, n), jnp.float32),
    )(x, w, b.reshape(1, n))


def _score_body(k_ref, q_ref, s_ref, sc_ref, mx_ref):
    t = k_ref[...] * q_ref[...]
    sc = jnp.dot(t, s_ref[...], preferred_element_type=jnp.float32)
    sc_ref[...] = sc
    cur = jnp.max(sc, axis=0, keepdims=True)

    @pl.when(pl.program_id(0) == 0)
    def _():
        mx_ref[...] = cur

    @pl.when(pl.program_id(0) != 0)
    def _():
        mx_ref[...] = jnp.maximum(mx_ref[...], cur)


def _scores(khat, qhat, smask, bm=2048):
    return pl.pallas_call(
        _score_body,
        grid=(EPAD // bm,),
        in_specs=[pl.BlockSpec((bm, C), lambda i: (i, 0)),
                  pl.BlockSpec((bm, C), lambda i: (i, 0)),
                  pl.BlockSpec((C, 8), lambda i: (0, 0))],
        out_specs=[pl.BlockSpec((bm, 8), lambda i: (i, 0)),
                   pl.BlockSpec((1, 8), lambda i: (0, 0))],
        out_shape=[jax.ShapeDtypeStruct((EPAD, 8), jnp.float32),
                   jax.ShapeDtypeStruct((1, 8), jnp.float32)],
    )(khat, qhat, smask)


def _msg_body(sc_ref, mx_ref, v_ref, r8_ref, p16_ref, msg_ref, e16_ref):
    e = jnp.exp(sc_ref[...] - mx_ref[...])
    e16_ref[...] = jnp.dot(e, p16_ref[...], preferred_element_type=jnp.float32)
    eexp = jnp.dot(e, r8_ref[...], preferred_element_type=jnp.float32)
    msg_ref[...] = v_ref[...] * eexp


def _messages(sc, mx, vhat, r8, p16, bm=2048):
    return pl.pallas_call(
        _msg_body,
        grid=(EPAD // bm,),
        in_specs=[pl.BlockSpec((bm, 8), lambda i: (i, 0)),
                  pl.BlockSpec((1, 8), lambda i: (0, 0)),
                  pl.BlockSpec((bm, C), lambda i: (i, 0)),
                  pl.BlockSpec((8, C), lambda i: (0, 0)),
                  pl.BlockSpec((8, 16), lambda i: (0, 0))],
        out_specs=[pl.BlockSpec((bm, C), lambda i: (i, 0)),
                   pl.BlockSpec((bm, 16), lambda i: (i, 0))],
        out_shape=[jax.ShapeDtypeStruct((EPAD, C), jnp.float32),
                   jax.ShapeDtypeStruct((EPAD, 16), jnp.float32)],
    )(sc, mx, vhat, r8, p16)


def _out_body(agg_ref, s_ref, p2_ref, wa_ref, ba_ref, xc_ref, xp_ref, o_ref):
    denom = jnp.maximum(jnp.dot(s_ref[...], p2_ref[...],
                                preferred_element_type=jnp.float32), 1e-30)
    u = agg_ref[...] / denom
    g = jax.nn.gelu(u)
    y = jnp.dot(g, wa_ref[...], preferred_element_type=jnp.float32)
    o_ref[...] = y + ba_ref[...] + xc_ref[...] * xp_ref[...]


def _out_proj(agg, s16, p2, wa, ba, xc, xprev, bm=1000):
    return pl.pallas_call(
        _out_body,
        grid=(N_NODE // bm,),
        in_specs=[pl.BlockSpec((bm, C), lambda i: (i, 0)),
                  pl.BlockSpec((bm, 16), lambda i: (i, 0)),
                  pl.BlockSpec((16, C), lambda i: (0, 0)),
                  pl.BlockSpec((C, C), lambda i: (0, 0)),
                  pl.BlockSpec((1, C), lambda i: (0, 0)),
                  pl.BlockSpec((1, C), lambda i: (0, 0)),
                  pl.BlockSpec((bm, C), lambda i: (i, 0))],
        out_specs=pl.BlockSpec((bm, C), lambda i: (i, 0)),
        out_shape=jax.ShapeDtypeStruct((N_NODE, C), jnp.float32),
    )(agg, s16, p2, wa, ba, xc, xprev)


def _heads_body(xu_ref, xi_ref, w0_ref, b0_ref, w1_ref, b1_ref,
                wh0_ref, bh0_ref, wh1_ref, bh1_ref, wh2_ref, bh2_ref,
                wv0_ref, bv0_ref, wv1_ref, bv1_ref,
                o0_ref, o1_ref, o2_ref, ov_ref, acc_ref):
    i = pl.program_id(0)
    su = jnp.sum(xu_ref[...], axis=0, keepdims=True)
    si = jnp.sum(xi_ref[...], axis=0, keepdims=True)
    cur = jnp.concatenate([su, si], axis=1)

    @pl.when(i == 0)
    def _():
        acc_ref[0:1, :] = cur

    @pl.when(i > 0)
    def _():
        acc_ref[0:1, :] = acc_ref[0:1, :] + cur

    @pl.when(i == pl.num_programs(0) - 1)
    def _():
        pooled = acc_ref[0:1, :] * (1.0 / N_NODE)

        def lin(v, w_ref, b_ref, act=False):
            y = jnp.dot(v, w_ref[...], preferred_element_type=jnp.float32)
            y = y + b_ref[...]
            return jnp.maximum(y, 0.0) if act else y

        h = lin(pooled, w0_ref, b0_ref, act=True)
        h = lin(h, w1_ref, b1_ref, act=True)
        o0_ref[...] = lin(h, wh0_ref, bh0_ref)
        o1_ref[...] = lin(h, wh1_ref, bh1_ref)
        o2_ref[...] = lin(h, wh2_ref, bh2_ref)
        vh = lin(h, wv0_ref, bv0_ref, act=True)
        ov_ref[...] = lin(vh, wv1_ref, bv1_ref)


def _pool_heads(xu, xi, params, bm=1000):
    sh = params["shared"]
    hd = params["heads"]
    vl = params["value"]
    full = lambda s: pl.BlockSpec(s, lambda i: tuple(0 for _ in s))
    args = [xu, xi,
            sh[0]["W"], sh[0]["b"].reshape(1, -1),
            sh[1]["W"], sh[1]["b"].reshape(1, -1),
            hd[0]["W"], hd[0]["b"].reshape(1, -1),
            hd[1]["W"], hd[1]["b"].reshape(1, -1),
            hd[2]["W"], hd[2]["b"].reshape(1, -1),
            vl[0]["W"], vl[0]["b"].reshape(1, -1),
            vl[1]["W"], vl[1]["b"].reshape(1, -1)]
    in_specs = [pl.BlockSpec((bm, C), lambda i: (i, 0)),
                pl.BlockSpec((bm, C), lambda i: (i, 0))]
    for a in args[2:]:
        in_specs.append(full(a.shape))
    out_shapes = [jax.ShapeDtypeStruct((1, 8), jnp.float32),
                  jax.ShapeDtypeStruct((1, 8), jnp.float32),
                  jax.ShapeDtypeStruct((1, 4), jnp.float32),
                  jax.ShapeDtypeStruct((1, 1), jnp.float32)]
    out_specs = [full(s.shape) for s in out_shapes]
    return pl.pallas_call(
        _heads_body,
        grid=(N_NODE // bm,),
        in_specs=in_specs,
        out_specs=out_specs,
        out_shape=out_shapes,
        scratch_shapes=[pltpu.VMEM((8, 2 * C), jnp.float32)],
    )(*args)


NCHG = EPW // CH
NCHS = EPT // CH
NBUF = 3


def _gather3_body(tk, ik, tq, iq, tv, okh, oqh, ovh,
                  i1, i2, r1, r2, s1, s2):
    wid = lax.axis_index("s") * NC + lax.axis_index("c")

    def phase(tA, iA, cA, oA, tB, iB, cB, oB):
        baseA = wid * EPW + cA * CH
        baseB = wid * EPW + cB * CH
        pltpu.sync_copy(iA.at[pl.ds(baseA, CH)], i1)
        pltpu.sync_copy(iB.at[pl.ds(baseB, CH)], i2)
        dA = pltpu.async_copy(tA.at[i1], r1, s1)
        dB = pltpu.async_copy(tB.at[i2], r2, s2)
        dA.wait()
        dB.wait()
        pltpu.sync_copy(r1, oA.at[pl.ds(baseA, CH)])
        pltpu.sync_copy(r2, oB.at[pl.ds(baseB, CH)])

    def body(g, carry):
        c0 = 2 * g
        c1 = 2 * g + 1
        phase(tk, ik, c0, okh, tq, iq, c0, oqh)
        phase(tv, ik, c0, ovh, tk, ik, c1, okh)
        phase(tq, iq, c1, oqh, tv, ik, c1, ovh)
        return carry

    lax.fori_loop(0, NCHG // 2, body, 0)


def _gather3(tk, ik, tq, iq, tv):
    k = pl.kernel(
        _gather3_body,
        out_type=(jax.ShapeDtypeStruct((EPAD, C), jnp.float32),
                  jax.ShapeDtypeStruct((EPAD, C), jnp.float32),
                  jax.ShapeDtypeStruct((EPAD, C), jnp.float32)),
        mesh=plsc.VectorSubcoreMesh(**_SC_MESH),
        scratch_types=[pltpu.VMEM((CH,), jnp.int32),
                       pltpu.VMEM((CH,), jnp.int32),
                       pltpu.VMEM((CH, C), jnp.float32),
                       pltpu.VMEM((CH, C), jnp.float32),
                       pltpu.SemaphoreType.DMA,
                       pltpu.SemaphoreType.DMA],
    )
    return k(tk, ik, tq, iq, tv)


SCH = 64


def _make_scatter_body(ncol):
    def _scatter_body(msg, dst, agg, dbuf, lbuf, mbuf, dbuf2, lbuf2, mbuf2,
                      sd0, sm0, sd1, sm1, tmsg):
        cid = lax.axis_index("c")
        sid = lax.axis_index("s")
        zv = jnp.zeros((LANES,), jnp.float32)

        for rr in range(SCH):
            for cc in range(ncol // LANES):
                mbuf[rr, pl.ds(cc * LANES, LANES)] = zv
        zr = TBL // NS
        for t in range(zr // SCH):
            pltpu.sync_copy(mbuf, tmsg.at[pl.ds(sid * zr + t * SCH, SCH)])
        rem = zr % SCH
        if rem:
            pltpu.sync_copy(mbuf.at[pl.ds(0, rem)],
                            tmsg.at[pl.ds(sid * zr + (zr // SCH) * SCH, rem)])
        plsc.subcore_barrier()

        lo = cid * HALF0
        halfc = HALF0 - (HALF0 - HALF1) * cid

        def body(g, carry):
            base0 = sid * EPT + (2 * g) * SCH
            base1 = sid * EPT + (2 * g + 1) * SCH
            d0 = pltpu.async_copy(dst.at[pl.ds(base0, SCH)], dbuf, sd0)
            m0 = pltpu.async_copy(msg.at[pl.ds(base0, SCH)], mbuf, sm0)
            d1 = pltpu.async_copy(dst.at[pl.ds(base1, SCH)], dbuf2, sd1)
            m1 = pltpu.async_copy(msg.at[pl.ds(base1, SCH)], mbuf2, sm1)
            d0.wait()
            m0.wait()
            for j in range(SCH // LANES):
                v = dbuf[pl.ds(j * LANES, LANES)]
                l = v - lo
                ok = (l >= 0) & (l < halfc)
                lbuf[pl.ds(j * LANES, LANES)] = jnp.where(ok, l, DUMP)
            pltpu.sync_copy(mbuf, tmsg.at[lbuf], add=True)
            d1.wait()
            m1.wait()
            for j in range(SCH // LANES):
                v = dbuf2[pl.ds(j * LANES, LANES)]
                l = v - lo
                ok = (l >= 0) & (l < halfc)
                lbuf2[pl.ds(j * LANES, LANES)] = jnp.where(ok, l, DUMP)
            pltpu.sync_copy(mbuf2, tmsg.at[lbuf2], add=True)
            return carry

        lax.fori_loop(0, EPT // SCH // 2, body, 0)
        plsc.subcore_barrier()

        nfull0 = HALF0 // SCH
        nfull1 = HALF1 // SCH
        nfullc = nfull0 - (nfull0 - nfull1) * cid

        def cpy(t, carry):
            ci = sid + NS * t

            @pl.when(ci < nfullc)
            def _():
                off = ci * SCH
                pltpu.sync_copy(tmsg.at[pl.ds(off, SCH)], mbuf)
                pltpu.sync_copy(mbuf, agg.at[pl.ds(lo + off, SCH)])
            return carry

        lax.fori_loop(0, (max(nfull0, nfull1) + NS - 1) // NS, cpy, 0)

        rem0 = HALF0 % SCH
        rem1 = HALF1 % SCH

        if rem0:
            @pl.when((sid == NS - 1) & (cid == 0))
            def _():
                roff = nfull0 * SCH
                pltpu.sync_copy(tmsg.at[pl.ds(roff, rem0)],
                                mbuf.at[pl.ds(0, rem0)])
                pltpu.sync_copy(mbuf.at[pl.ds(0, rem0)],
                                agg.at[pl.ds(roff, rem0)])

        if rem1:
            @pl.when((sid == NS - 1) & (cid == 1))
            def _():
                roff = nfull1 * SCH
                pltpu.sync_copy(tmsg.at[pl.ds(roff, rem1)],
                                mbuf.at[pl.ds(0, rem1)])
                pltpu.sync_copy(mbuf.at[pl.ds(0, rem1)],
                                agg.at[pl.ds(HALF0 + roff, rem1)])

    return _scatter_body


def _scatter(rows, dst, ncol):
    k = pl.kernel(
        _make_scatter_body(ncol),
        out_type=jax.ShapeDtypeStruct((N_NODE, ncol), jnp.float32),
        mesh=plsc.VectorSubcoreMesh(**_SC_MESH),
        scratch_types=[pltpu.VMEM((SCH,), jnp.int32),
                       pltpu.VMEM((SCH,), jnp.int32),
                       pltpu.VMEM((SCH, ncol), jnp.float32),
                       pltpu.VMEM((SCH,), jnp.int32),
                       pltpu.VMEM((SCH,), jnp.int32),
                       pltpu.VMEM((SCH, ncol), jnp.float32),
                       pltpu.SemaphoreType.DMA,
                       pltpu.SemaphoreType.DMA,
                       pltpu.SemaphoreType.DMA,
                       pltpu.SemaphoreType.DMA,
                       pltpu.VMEM_SHARED((TBL, ncol), jnp.float32)],
    )
    return k(rows, dst)


def _block_diag(a):
    out = jnp.zeros((C, C), dtype=jnp.float32)
    for h in range(H):
        out = lax.dynamic_update_slice(out, a[h], (h * D, h * D))
    return out


def kernel(x_user, x_item, edge_index_user_item, edge_index_item_user, params):
    edge = {"u2i": edge_index_user_item, "i2u": edge_index_item_user}

    smask = np.zeros((C, 8), np.float32)
    for h in range(H):
        smask[h * D:(h + 1) * D, h] = 1.0
    smask = jnp.asarray(smask)
    r8 = np.zeros((8, C), np.float32)
    for h in range(H):
        r8[h, h * D:(h + 1) * D] = 1.0
    r8 = jnp.asarray(r8)
    p16 = np.zeros((8, 16), np.float32)
    for h in range(H):
        p16[h, h] = 1.0
    p16 = jnp.asarray(p16)
    p2 = np.zeros((16, C), np.float32)
    for h in range(H):
        p2[h, h * D:(h + 1) * D] = 1.0
    p2 = jnp.asarray(p2)

    pad_i = jnp.zeros((EPAD - E_REL,), jnp.int32)
    pad_d = jnp.full((EPAD - E_REL,), N_NODE, jnp.int32)
    src_pad = {r: jnp.concatenate([edge[r][0], pad_i]) for (_, r, _) in RELATIONS}
    dst_pad = {r: jnp.concatenate([edge[r][1], pad_d]) for (_, r, _) in RELATIONS}

    x = {"user": _mm(x_user, params["lin_in"]["user"]["W"],
                     params["lin_in"]["user"]["b"], act="relu"),
         "item": _mm(x_item, params["lin_in"]["item"]["W"],
                     params["lin_in"]["item"]["b"], act="relu")}

    for layer in params["layers"]:
        newx = {}
        for (s, r, d) in RELATIONS:
            rp = layer["rel"][r]
            colscale = jnp.repeat(rp["p_rel"], D) * (1.0 / np.sqrt(D))
            ak = _block_diag(rp["a_rel"]) * colscale[None, :]
            mv = _block_diag(rp["m_rel"])
            wk = layer["k"][s]["W"] @ ak
            bk = layer["k"][s]["b"] @ ak
            wv = layer["v"][s]["W"] @ mv
            bv = layer["v"][s]["b"] @ mv

            krel = _mm(x[s], wk, bk)
            vrel = _mm(x[s], wv, bv)
            q = _mm(x[d], layer["q"][d]["W"], layer["q"][d]["b"])

            khat, qhat, vhat = _gather3(krel, src_pad[r], q, dst_pad[r],
                                        vrel)

            sc, mx = _scores(khat, qhat, smask)
            msgs, e16 = _messages(sc, mx, vhat, r8, p16)
            agg = _scatter(msgs, dst_pad[r], C)
            s16 = _scatter(e16, dst_pad[r], 16)

            beta = jax.nn.sigmoid(layer["skip"][d])
            wa = layer["a"][d]["W"] * beta
            ba = (layer["a"][d]["b"] * beta).reshape(1, C)
            xc = jnp.full((1, C), 1.0 - beta, jnp.float32)
            newx[d] = _out_proj(agg, s16, p2, wa, ba, xc, x[d])
        x = newx

    o0, o1, o2, ov = _pool_heads(x["user"], x["item"], params)
    return (o0, o1, o2, ov)


# speedup vs baseline: 19.0613x; 1.0411x over previous
import functools

import numpy as np
import jax
import jax.numpy as jnp
from jax import lax
from jax.experimental import pallas as pl
from jax.experimental.pallas import tpu as pltpu
from jax.experimental.pallas import tpu_sc as plsc

N_USER = 25000
N_ITEM = 25000
N_NODE = 25000
E_REL = 200000
C = 128
H = 4
D = C // H
NODE_TYPES = ["user", "item"]
RELATIONS = [("user", "u2i", "item"), ("item", "i2u", "user")]

NC = 2
NS = 16
NW = NC * NS
LANES = 16

EPAD = 204800
EPW = EPAD // NW
EPT = EPAD // NS
CH = 128
HALF0 = 12504
HALF1 = N_NODE - HALF0
DUMP = 12520
TBL = 12544

_SC_MESH = dict(core_axis_name="c", subcore_axis_name="s",
                num_cores=NC, num_subcores=NS)


def _mm_body(act, x_ref, w_ref, b_ref, o_ref):
    y = jnp.dot(x_ref[...], w_ref[...], preferred_element_type=jnp.float32)
    y = y + b_ref[...]
    if act == "relu":
        y = jnp.maximum(y, 0.0)
    o_ref[...] = y


def _mm(x, w, b, act=None, bm=1000):
    m, k = x.shape
    n = w.shape[1]
    return pl.pallas_call(
        functools.partial(_mm_body, act),
        grid=(m // bm,),
        in_specs=[pl.BlockSpec((bm, k), lambda i: (i, 0)),
                  pl.BlockSpec((k, n), lambda i: (0, 0)),
                  pl.BlockSpec((1, n), lambda i: (0, 0))],
        out_specs=pl.BlockSpec((bm, n), lambda i: (i, 0)),
        out_shape=jax.ShapeDtypeStruct((m, n), jnp.float32),
    )(x, w, b.reshape(1, n))


def _score_body(k_ref, q_ref, s_ref, sc_ref, mx_ref):
    t = k_ref[...] * q_ref[...]
    sc = jnp.dot(t, s_ref[...], preferred_element_type=jnp.float32)
    sc_ref[...] = sc
    cur = jnp.max(sc, axis=0, keepdims=True)

    @pl.when(pl.program_id(0) == 0)
    def _():
        mx_ref[...] = cur

    @pl.when(pl.program_id(0) != 0)
    def _():
        mx_ref[...] = jnp.maximum(mx_ref[...], cur)


def _scores(khat, qhat, smask, bm=2048):
    return pl.pallas_call(
        _score_body,
        grid=(EPAD // bm,),
        in_specs=[pl.BlockSpec((bm, C), lambda i: (i, 0)),
                  pl.BlockSpec((bm, C), lambda i: (i, 0)),
                  pl.BlockSpec((C, 8), lambda i: (0, 0))],
        out_specs=[pl.BlockSpec((bm, 8), lambda i: (i, 0)),
                   pl.BlockSpec((1, 8), lambda i: (0, 0))],
        out_shape=[jax.ShapeDtypeStruct((EPAD, 8), jnp.float32),
                   jax.ShapeDtypeStruct((1, 8), jnp.float32)],
    )(khat, qhat, smask)


def _msg_body(sc_ref, mx_ref, v_ref, r8_ref, p16_ref, msg_ref, e16_ref):
    e = jnp.exp(sc_ref[...] - mx_ref[...])
    e16_ref[...] = jnp.dot(e, p16_ref[...], preferred_element_type=jnp.float32)
    eexp = jnp.dot(e, r8_ref[...], preferred_element_type=jnp.float32)
    msg_ref[...] = v_ref[...] * eexp


def _messages(sc, mx, vhat, r8, p16, bm=2048):
    return pl.pallas_call(
        _msg_body,
        grid=(EPAD // bm,),
        in_specs=[pl.BlockSpec((bm, 8), lambda i: (i, 0)),
                  pl.BlockSpec((1, 8), lambda i: (0, 0)),
                  pl.BlockSpec((bm, C), lambda i: (i, 0)),
                  pl.BlockSpec((8, C), lambda i: (0, 0)),
                  pl.BlockSpec((8, 16), lambda i: (0, 0))],
        out_specs=[pl.BlockSpec((bm, C), lambda i: (i, 0)),
                   pl.BlockSpec((bm, 16), lambda i: (i, 0))],
        out_shape=[jax.ShapeDtypeStruct((EPAD, C), jnp.float32),
                   jax.ShapeDtypeStruct((EPAD, 16), jnp.float32)],
    )(sc, mx, vhat, r8, p16)


def _out_body(agg_ref, s_ref, p2_ref, wa_ref, ba_ref, xc_ref, xp_ref, o_ref):
    denom = jnp.maximum(jnp.dot(s_ref[...], p2_ref[...],
                                preferred_element_type=jnp.float32), 1e-30)
    u = agg_ref[...] / denom
    g = jax.nn.gelu(u)
    y = jnp.dot(g, wa_ref[...], preferred_element_type=jnp.float32)
    o_ref[...] = y + ba_ref[...] + xc_ref[...] * xp_ref[...]


def _out_proj(agg, s16, p2, wa, ba, xc, xprev, bm=1000):
    return pl.pallas_call(
        _out_body,
        grid=(N_NODE // bm,),
        in_specs=[pl.BlockSpec((bm, C), lambda i: (i, 0)),
                  pl.BlockSpec((bm, 16), lambda i: (i, 0)),
                  pl.BlockSpec((16, C), lambda i: (0, 0)),
                  pl.BlockSpec((C, C), lambda i: (0, 0)),
                  pl.BlockSpec((1, C), lambda i: (0, 0)),
                  pl.BlockSpec((1, C), lambda i: (0, 0)),
                  pl.BlockSpec((bm, C), lambda i: (i, 0))],
        out_specs=pl.BlockSpec((bm, C), lambda i: (i, 0)),
        out_shape=jax.ShapeDtypeStruct((N_NODE, C), jnp.float32),
    )(agg, s16, p2, wa, ba, xc, xprev)


def _heads_body(xu_ref, xi_ref, w0_ref, b0_ref, w1_ref, b1_ref,
                wh0_ref, bh0_ref, wh1_ref, bh1_ref, wh2_ref, bh2_ref,
                wv0_ref, bv0_ref, wv1_ref, bv1_ref,
                o0_ref, o1_ref, o2_ref, ov_ref, acc_ref):
    i = pl.program_id(0)
    su = jnp.sum(xu_ref[...], axis=0, keepdims=True)
    si = jnp.sum(xi_ref[...], axis=0, keepdims=True)
    cur = jnp.concatenate([su, si], axis=1)

    @pl.when(i == 0)
    def _():
        acc_ref[0:1, :] = cur

    @pl.when(i > 0)
    def _():
        acc_ref[0:1, :] = acc_ref[0:1, :] + cur

    @pl.when(i == pl.num_programs(0) - 1)
    def _():
        pooled = acc_ref[0:1, :] * (1.0 / N_NODE)

        def lin(v, w_ref, b_ref, act=False):
            y = jnp.dot(v, w_ref[...], preferred_element_type=jnp.float32)
            y = y + b_ref[...]
            return jnp.maximum(y, 0.0) if act else y

        h = lin(pooled, w0_ref, b0_ref, act=True)
        h = lin(h, w1_ref, b1_ref, act=True)
        o0_ref[...] = lin(h, wh0_ref, bh0_ref)
        o1_ref[...] = lin(h, wh1_ref, bh1_ref)
        o2_ref[...] = lin(h, wh2_ref, bh2_ref)
        vh = lin(h, wv0_ref, bv0_ref, act=True)
        ov_ref[...] = lin(vh, wv1_ref, bv1_ref)


def _pool_heads(xu, xi, params, bm=1000):
    sh = params["shared"]
    hd = params["heads"]
    vl = params["value"]
    full = lambda s: pl.BlockSpec(s, lambda i: tuple(0 for _ in s))
    args = [xu, xi,
            sh[0]["W"], sh[0]["b"].reshape(1, -1),
            sh[1]["W"], sh[1]["b"].reshape(1, -1),
            hd[0]["W"], hd[0]["b"].reshape(1, -1),
            hd[1]["W"], hd[1]["b"].reshape(1, -1),
            hd[2]["W"], hd[2]["b"].reshape(1, -1),
            vl[0]["W"], vl[0]["b"].reshape(1, -1),
            vl[1]["W"], vl[1]["b"].reshape(1, -1)]
    in_specs = [pl.BlockSpec((bm, C), lambda i: (i, 0)),
                pl.BlockSpec((bm, C), lambda i: (i, 0))]
    for a in args[2:]:
        in_specs.append(full(a.shape))
    out_shapes = [jax.ShapeDtypeStruct((1, 8), jnp.float32),
                  jax.ShapeDtypeStruct((1, 8), jnp.float32),
                  jax.ShapeDtypeStruct((1, 4), jnp.float32),
                  jax.ShapeDtypeStruct((1, 1), jnp.float32)]
    out_specs = [full(s.shape) for s in out_shapes]
    return pl.pallas_call(
        _heads_body,
        grid=(N_NODE // bm,),
        in_specs=in_specs,
        out_specs=out_specs,
        out_shape=out_shapes,
        scratch_shapes=[pltpu.VMEM((8, 2 * C), jnp.float32)],
    )(*args)


NCHG = EPW // CH
NCHS = EPT // CH
NBUF = 3


def _gather3_body(tk, ik, tq, iq, tv, okh, oqh, ovh,
                  i1, i2, r1, r2, r3, r4, s1, s2, w1, w2, w3, w4):
    wid = lax.axis_index("s") * NC + lax.axis_index("c")

    def gpair(tA, iA, cA, rA, tB, iB, cB, rB):
        baseA = wid * EPW + cA * CH
        baseB = wid * EPW + cB * CH
        pltpu.sync_copy(iA.at[pl.ds(baseA, CH)], i1)
        pltpu.sync_copy(iB.at[pl.ds(baseB, CH)], i2)
        dA = pltpu.async_copy(tA.at[i1], rA, s1)
        dB = pltpu.async_copy(tB.at[i2], rB, s2)
        dA.wait()
        dB.wait()

    def body(g, carry):
        c0 = 2 * g
        c1 = 2 * g + 1
        b0 = wid * EPW + c0 * CH
        b1 = wid * EPW + c1 * CH
        gpair(tk, ik, c0, r1, tq, iq, c0, r2)
        wa = pltpu.async_copy(r1, okh.at[pl.ds(b0, CH)], w1)
        wb = pltpu.async_copy(r2, oqh.at[pl.ds(b0, CH)], w2)
        gpair(tv, ik, c0, r3, tk, ik, c1, r4)
        wc = pltpu.async_copy(r3, ovh.at[pl.ds(b0, CH)], w3)
        wd = pltpu.async_copy(r4, okh.at[pl.ds(b1, CH)], w4)
        wa.wait()
        wb.wait()
        gpair(tq, iq, c1, r1, tv, ik, c1, r2)
        we = pltpu.async_copy(r1, oqh.at[pl.ds(b1, CH)], w1)
        wf = pltpu.async_copy(r2, ovh.at[pl.ds(b1, CH)], w2)
        wc.wait()
        wd.wait()
        we.wait()
        wf.wait()
        return carry

    lax.fori_loop(0, NCHG // 2, body, 0)


def _gather3(tk, ik, tq, iq, tv):
    k = pl.kernel(
        _gather3_body,
        out_type=(jax.ShapeDtypeStruct((EPAD, C), jnp.float32),
                  jax.ShapeDtypeStruct((EPAD, C), jnp.float32),
                  jax.ShapeDtypeStruct((EPAD, C), jnp.float32)),
        mesh=plsc.VectorSubcoreMesh(**_SC_MESH),
        scratch_types=[pltpu.VMEM((CH,), jnp.int32),
                       pltpu.VMEM((CH,), jnp.int32),
                       pltpu.VMEM((CH, C), jnp.float32),
                       pltpu.VMEM((CH, C), jnp.float32),
                       pltpu.VMEM((CH, C), jnp.float32),
                       pltpu.VMEM((CH, C), jnp.float32),
                       pltpu.SemaphoreType.DMA,
                       pltpu.SemaphoreType.DMA,
                       pltpu.SemaphoreType.DMA,
                       pltpu.SemaphoreType.DMA,
                       pltpu.SemaphoreType.DMA,
                       pltpu.SemaphoreType.DMA],
    )
    return k(tk, ik, tq, iq, tv)


SCH = 64


def _make_scatter_body(ncol):
    def _scatter_body(msg, dst, agg, dbuf, lbuf, mbuf, dbuf2, lbuf2, mbuf2,
                      sd0, sm0, sd1, sm1, tmsg):
        cid = lax.axis_index("c")
        sid = lax.axis_index("s")
        zv = jnp.zeros((LANES,), jnp.float32)

        for rr in range(SCH):
            for cc in range(ncol // LANES):
                mbuf[rr, pl.ds(cc * LANES, LANES)] = zv
        zr = TBL // NS
        for t in range(zr // SCH):
            pltpu.sync_copy(mbuf, tmsg.at[pl.ds(sid * zr + t * SCH, SCH)])
        rem = zr % SCH
        if rem:
            pltpu.sync_copy(mbuf.at[pl.ds(0, rem)],
                            tmsg.at[pl.ds(sid * zr + (zr // SCH) * SCH, rem)])
        plsc.subcore_barrier()

        lo = cid * HALF0
        halfc = HALF0 - (HALF0 - HALF1) * cid

        def body(g, carry):
            base0 = sid * EPT + (2 * g) * SCH
            base1 = sid * EPT + (2 * g + 1) * SCH
            d0 = pltpu.async_copy(dst.at[pl.ds(base0, SCH)], dbuf, sd0)
            m0 = pltpu.async_copy(msg.at[pl.ds(base0, SCH)], mbuf, sm0)
            d1 = pltpu.async_copy(dst.at[pl.ds(base1, SCH)], dbuf2, sd1)
            m1 = pltpu.async_copy(msg.at[pl.ds(base1, SCH)], mbuf2, sm1)
            d0.wait()
            m0.wait()
            for j in range(SCH // LANES):
                v = dbuf[pl.ds(j * LANES, LANES)]
                l = v - lo
                ok = (l >= 0) & (l < halfc)
                lbuf[pl.ds(j * LANES, LANES)] = jnp.where(ok, l, DUMP)
            pltpu.sync_copy(mbuf, tmsg.at[lbuf], add=True)
            d1.wait()
            m1.wait()
            for j in range(SCH // LANES):
                v = dbuf2[pl.ds(j * LANES, LANES)]
                l = v - lo
                ok = (l >= 0) & (l < halfc)
                lbuf2[pl.ds(j * LANES, LANES)] = jnp.where(ok, l, DUMP)
            pltpu.sync_copy(mbuf2, tmsg.at[lbuf2], add=True)
            return carry

        lax.fori_loop(0, EPT // SCH // 2, body, 0)
        plsc.subcore_barrier()

        nfull0 = HALF0 // SCH
        nfull1 = HALF1 // SCH
        nfullc = nfull0 - (nfull0 - nfull1) * cid

        def cpy(t, carry):
            ci = sid + NS * t

            @pl.when(ci < nfullc)
            def _():
                off = ci * SCH
                pltpu.sync_copy(tmsg.at[pl.ds(off, SCH)], mbuf)
                pltpu.sync_copy(mbuf, agg.at[pl.ds(lo + off, SCH)])
            return carry

        lax.fori_loop(0, (max(nfull0, nfull1) + NS - 1) // NS, cpy, 0)

        rem0 = HALF0 % SCH
        rem1 = HALF1 % SCH

        if rem0:
            @pl.when((sid == NS - 1) & (cid == 0))
            def _():
                roff = nfull0 * SCH
                pltpu.sync_copy(tmsg.at[pl.ds(roff, rem0)],
                                mbuf.at[pl.ds(0, rem0)])
                pltpu.sync_copy(mbuf.at[pl.ds(0, rem0)],
                                agg.at[pl.ds(roff, rem0)])

        if rem1:
            @pl.when((sid == NS - 1) & (cid == 1))
            def _():
                roff = nfull1 * SCH
                pltpu.sync_copy(tmsg.at[pl.ds(roff, rem1)],
                                mbuf.at[pl.ds(0, rem1)])
                pltpu.sync_copy(mbuf.at[pl.ds(0, rem1)],
                                agg.at[pl.ds(HALF0 + roff, rem1)])

    return _scatter_body


def _scatter(rows, dst, ncol):
    k = pl.kernel(
        _make_scatter_body(ncol),
        out_type=jax.ShapeDtypeStruct((N_NODE, ncol), jnp.float32),
        mesh=plsc.VectorSubcoreMesh(**_SC_MESH),
        scratch_types=[pltpu.VMEM((SCH,), jnp.int32),
                       pltpu.VMEM((SCH,), jnp.int32),
                       pltpu.VMEM((SCH, ncol), jnp.float32),
                       pltpu.VMEM((SCH,), jnp.int32),
                       pltpu.VMEM((SCH,), jnp.int32),
                       pltpu.VMEM((SCH, ncol), jnp.float32),
                       pltpu.SemaphoreType.DMA,
                       pltpu.SemaphoreType.DMA,
                       pltpu.SemaphoreType.DMA,
                       pltpu.SemaphoreType.DMA,
                       pltpu.VMEM_SHARED((TBL, ncol), jnp.float32)],
    )
    return k(rows, dst)


def _block_diag(a):
    out = jnp.zeros((C, C), dtype=jnp.float32)
    for h in range(H):
        out = lax.dynamic_update_slice(out, a[h], (h * D, h * D))
    return out


def kernel(x_user, x_item, edge_index_user_item, edge_index_item_user, params):
    edge = {"u2i": edge_index_user_item, "i2u": edge_index_item_user}

    smask = np.zeros((C, 8), np.float32)
    for h in range(H):
        smask[h * D:(h + 1) * D, h] = 1.0
    smask = jnp.asarray(smask)
    r8 = np.zeros((8, C), np.float32)
    for h in range(H):
        r8[h, h * D:(h + 1) * D] = 1.0
    r8 = jnp.asarray(r8)
    p16 = np.zeros((8, 16), np.float32)
    for h in range(H):
        p16[h, h] = 1.0
    p16 = jnp.asarray(p16)
    p2 = np.zeros((16, C), np.float32)
    for h in range(H):
        p2[h, h * D:(h + 1) * D] = 1.0
    p2 = jnp.asarray(p2)

    pad_i = jnp.zeros((EPAD - E_REL,), jnp.int32)
    pad_d = jnp.full((EPAD - E_REL,), N_NODE, jnp.int32)
    src_pad = {r: jnp.concatenate([edge[r][0], pad_i]) for (_, r, _) in RELATIONS}
    dst_pad = {r: jnp.concatenate([edge[r][1], pad_d]) for (_, r, _) in RELATIONS}

    x = {"user": _mm(x_user, params["lin_in"]["user"]["W"],
                     params["lin_in"]["user"]["b"], act="relu"),
         "item": _mm(x_item, params["lin_in"]["item"]["W"],
                     params["lin_in"]["item"]["b"], act="relu")}

    for layer in params["layers"]:
        newx = {}
        for (s, r, d) in RELATIONS:
            rp = layer["rel"][r]
            colscale = jnp.repeat(rp["p_rel"], D) * (1.0 / np.sqrt(D))
            ak = _block_diag(rp["a_rel"]) * colscale[None, :]
            mv = _block_diag(rp["m_rel"])
            wk = layer["k"][s]["W"] @ ak
            bk = layer["k"][s]["b"] @ ak
            wv = layer["v"][s]["W"] @ mv
            bv = layer["v"][s]["b"] @ mv

            krel = _mm(x[s], wk, bk)
            vrel = _mm(x[s], wv, bv)
            q = _mm(x[d], layer["q"][d]["W"], layer["q"][d]["b"])

            khat, qhat, vhat = _gather3(krel, src_pad[r], q, dst_pad[r],
                                        vrel)

            sc, mx = _scores(khat, qhat, smask)
            msgs, e16 = _messages(sc, mx, vhat, r8, p16)
            agg = _scatter(msgs, dst_pad[r], C)
            s16 = _scatter(e16, dst_pad[r], 16)

            beta = jax.nn.sigmoid(layer["skip"][d])
            wa = layer["a"][d]["W"] * beta
            ba = (layer["a"][d]["b"] * beta).reshape(1, C)
            xc = jnp.full((1, C), 1.0 - beta, jnp.float32)
            newx[d] = _out_proj(agg, s16, p2, wa, ba, xc, x[d])
        x = newx

    o0, o1, o2, ov = _pool_heads(x["user"], x["item"], params)
    return (o0, o1, o2, ov)
